# baseline (speedup 1.0000x reference)
import functools

import jax
import jax.numpy as jnp
from jax.experimental import pallas as pl
from jax.experimental.pallas import tpu as pltpu


def _pick_tile(n, candidates):
    for t in candidates:
        if n % t == 0:
            return t
    return 128


def _fused_kernel(adj_ref, x0_ref, feats_ref, w_ref, b_ref,
                  items_ref, v_ref, t_ref, a16_ref, x1c_ref, x0c_ref,
                  *, ta, tf, emb, n_a, n_cached):
    s = pl.program_id(0)
    sub = 256 if ta % 256 == 0 else 128

    @pl.when(s == 0)
    def _():
        x0c_ref[...] = x0_ref[...].astype(jnp.bfloat16)

    @pl.when(s < n_a - 1)
    def _():
        for h in range(ta // sub):
            row = s * ta + h * sub
            a16_ref[pl.ds(row, sub), :] = (
                adj_ref[pl.ds(h * sub, sub), :].astype(jnp.bfloat16))
            x1c_ref[pl.ds(row, sub), :] = jnp.dot(
                a16_ref[pl.ds(row, sub), :], x0c_ref[...],
                preferred_element_type=jnp.float32).astype(jnp.bfloat16)

    @pl.when(s == n_a - 1)
    def _():
        for h in range(ta // 128):
            row = s * ta + h * 128
            x1c_ref[pl.ds(row, 128), :] = jnp.dot(
                adj_ref[pl.ds(h * 128, 128), :].astype(jnp.bfloat16),
                x0c_ref[...],
                preferred_element_type=jnp.float32).astype(jnp.bfloat16)

    @pl.when(s >= n_a)
    def _():
        proj = jnp.dot(feats_ref[...], w_ref[...],
                       preferred_element_type=jnp.float32) + b_ref[...]
        v_ref[...] = proj[:, :emb]
        t_ref[...] = proj[:, emb:]

        row = (s - n_a) * tf
        @pl.when(row < n_cached)
        def _():
            items_ref[pl.ds(row, tf), :] = jnp.dot(
                a16_ref[pl.ds(row, tf), :], x1c_ref[...],
                preferred_element_type=jnp.float32)

        @pl.when(row >= n_cached)
        def _():
            for h in range(tf // 128):
                items_ref[pl.ds(row + h * 128, 128), :] = jnp.dot(
                    adj_ref[pl.ds(row - n_cached + h * 128, 128),
                            :].astype(jnp.bfloat16),
                    x1c_ref[...], preferred_element_type=jnp.float32)


def kernel(mAdj, itemEmbds, featsPadded, wBlk, bCat):
    n, emb = itemEmbds.shape
    k_pad = featsPadded.shape[1]
    out_w = wBlk.shape[1]

    ta = _pick_tile(n, (512, 256, 128))
    tf = _pick_tile(n, (256, 128))
    n_a = n // ta
    n_f = n // tf
    n_cached = n - ta
    last_f = n_f - 1

    flops = 2 * (2 * n * n * emb + n * k_pad * out_w)
    bytes_accessed = 4 * (n * n + n * k_pad + n * emb
                          + k_pad * out_w + out_w + 3 * n * emb)

    items, v, t = pl.pallas_call(
        functools.partial(_fused_kernel, ta=ta, tf=tf, emb=emb,
                          n_a=n_a, n_cached=n_cached),
        out_shape=[jax.ShapeDtypeStruct((n, emb), jnp.float32),
                   jax.ShapeDtypeStruct((n, emb), jnp.float32),
                   jax.ShapeDtypeStruct((n, emb), jnp.float32)],
        grid_spec=pltpu.PrefetchScalarGridSpec(
            num_scalar_prefetch=0,
            grid=(n_a + n_f,),
            in_specs=[
                pl.BlockSpec((ta, n),
                             lambda s: (jnp.minimum(s, n_a - 1), 0)),
                pl.BlockSpec((n, emb), lambda s: (0, 0)),
                pl.BlockSpec((tf, k_pad),
                             lambda s: (jnp.clip(s - n_a, 0, last_f), 0)),
                pl.BlockSpec((k_pad, out_w), lambda s: (0, 0)),
                pl.BlockSpec((1, out_w), lambda s: (0, 0)),
            ],
            out_specs=[
                pl.BlockSpec((n, emb), lambda s: (0, 0)),
                pl.BlockSpec((tf, emb),
                             lambda s: (jnp.clip(s - n_a, 0, last_f), 0)),
                pl.BlockSpec((tf, emb),
                             lambda s: (jnp.clip(s - n_a, 0, last_f), 0)),
            ],
            scratch_shapes=[pltpu.VMEM((n - ta, n), jnp.bfloat16),
                            pltpu.VMEM((n, emb), jnp.bfloat16),
                            pltpu.VMEM((n, emb), jnp.bfloat16)]),
        compiler_params=pltpu.CompilerParams(
            dimension_semantics=("arbitrary",),
            vmem_limit_bytes=67000000),
        cost_estimate=pl.CostEstimate(flops=flops, transcendentals=0,
                                      bytes_accessed=bytes_accessed),
    )(mAdj, itemEmbds, featsPadded, wBlk, bCat)

    return items, v, t


# speedup vs baseline: 1.0115x; 1.0115x over previous
import functools

import jax
import jax.numpy as jnp
from jax.experimental import pallas as pl
from jax.experimental.pallas import tpu as pltpu


def _pick_tile(n, candidates):
    for t in candidates:
        if n % t == 0:
            return t
    return 128


def _fused_kernel(adj_ref, x0_ref, feats_ref, w_ref, b_ref,
                  items_ref, v_ref, t_ref, a16_ref, x1c_ref, x0c_ref,
                  *, tm, emb, n_s):
    s = pl.program_id(0)

    @pl.when(s == 0)
    def _():
        x0c_ref[...] = x0_ref[...].astype(jnp.bfloat16)

    @pl.when(s < n_s)
    def _():
        a16_ref[pl.ds(s * tm, tm), :] = adj_ref[...].astype(jnp.bfloat16)
        x1c_ref[pl.ds(s * tm, tm), :] = jnp.dot(
            a16_ref[pl.ds(s * tm, tm), :], x0c_ref[...],
            preferred_element_type=jnp.float32).astype(jnp.bfloat16)
        proj = jnp.dot(feats_ref[...], w_ref[...],
                       preferred_element_type=jnp.float32) + b_ref[...]
        v_ref[...] = proj[:, :emb]
        t_ref[...] = proj[:, emb:]

    @pl.when(s == n_s)
    def _():
        def _chunk(c, carry):
            items_ref[pl.ds(c * tm, tm), :] = jnp.dot(
                a16_ref[pl.ds(c * tm, tm), :], x1c_ref[...],
                preferred_element_type=jnp.float32)
            return carry

        jax.lax.fori_loop(0, n_s, _chunk, 0)


def kernel(mAdj, itemEmbds, featsPadded, wBlk, bCat):
    n, emb = itemEmbds.shape
    k_pad = featsPadded.shape[1]
    out_w = wBlk.shape[1]

    tm = _pick_tile(n, (256, 128))
    n_s = n // tm
    last = n_s - 1

    flops = 2 * (2 * n * n * emb + n * k_pad * out_w)
    bytes_accessed = 4 * (n * n + n * k_pad + n * emb
                          + k_pad * out_w + out_w + 3 * n * emb)

    items, v, t = pl.pallas_call(
        functools.partial(_fused_kernel, tm=tm, emb=emb, n_s=n_s),
        out_shape=[jax.ShapeDtypeStruct((n, emb), jnp.float32),
                   jax.ShapeDtypeStruct((n, emb), jnp.float32),
                   jax.ShapeDtypeStruct((n, emb), jnp.float32)],
        grid_spec=pltpu.PrefetchScalarGridSpec(
            num_scalar_prefetch=0,
            grid=(n_s + 1,),
            in_specs=[
                pl.BlockSpec((tm, n),
                             lambda s: (jnp.minimum(s, last), 0)),
                pl.BlockSpec((n, emb), lambda s: (0, 0)),
                pl.BlockSpec((tm, k_pad),
                             lambda s: (jnp.minimum(s, last), 0)),
                pl.BlockSpec((k_pad, out_w), lambda s: (0, 0)),
                pl.BlockSpec((1, out_w), lambda s: (0, 0)),
            ],
            out_specs=[
                pl.BlockSpec((n, emb), lambda s: (0, 0)),
                pl.BlockSpec((tm, emb), lambda s: (jnp.minimum(s, last), 0)),
                pl.BlockSpec((tm, emb), lambda s: (jnp.minimum(s, last), 0)),
            ],
            scratch_shapes=[pltpu.VMEM((n, n), jnp.bfloat16),
                            pltpu.VMEM((n, emb), jnp.bfloat16),
                            pltpu.VMEM((n, emb), jnp.bfloat16)]),
        compiler_params=pltpu.CompilerParams(
            dimension_semantics=("arbitrary",)),
        cost_estimate=pl.CostEstimate(flops=flops, transcendentals=0,
                                      bytes_accessed=bytes_accessed),
    )(mAdj, itemEmbds, featsPadded, wBlk, bCat)

    return items, v, t
